# Initial kernel scaffold; baseline (speedup 1.0000x reference)
import jax
import jax.numpy as jnp
from jax.experimental import pallas as pl


def kernel(xe1, xe2, ee1, ee2, w1, b1, w2, b2, bn_g, bn_b, featW, featB, outW1, outB1, outW2, outB2, labelE, x, edge_index, edge_attr, batch, cluster_idx):
    raise NotImplementedError("write your pallas kernel here")


# speedup vs baseline: 4.9309x; 4.9309x over previous
import functools

import jax
import jax.numpy as jnp
from jax import lax
from jax.experimental import pallas as pl
from jax.experimental.pallas import tpu as pltpu
from jax.experimental.pallas import tpu_sc as plsc

_N = 10000
_E = 160000
_L = 5
_D = 300
_FD = 512
_G = 128
_DH = 150
_H = 160
_NS = 16
_NC = 2
_RPT = 632
_NP = _NS * _RPT
_TRASH = _N
_CHUNK = 128
_EP = _NS * 80 * _CHUNK
_RB = 2000
_EB = 4096


@functools.cache
def _sc_mesh():
    return plsc.VectorSubcoreMesh(
        core_axis_name="c", subcore_axis_name="s",
        num_cores=_NC, num_subcores=_NS,
    )


def _sc_scatter_body(habf, src2, dstp, zra, out, idx_v, didx_v, rows_v, sem,
                     acc):
    c = lax.axis_index("c")
    s = lax.axis_index("s")
    r0 = pl.multiple_of(s * _RPT, 8)
    pltpu.sync_copy(zra, acc.at[pl.ds(r0, _RPT)])
    plsc.subcore_barrier()

    tbase = s * (_EP // _NS)

    def body(i, carry):
        base = tbase + i * _CHUNK
        pltpu.sync_copy(src2.at[c, pl.ds(base, _CHUNK)], idx_v)
        pltpu.sync_copy(dstp.at[pl.ds(base, _CHUNK)], didx_v)
        pltpu.async_copy(habf.at[idx_v], rows_v, sem).wait()
        pltpu.sync_copy(rows_v, acc.at[didx_v], add=True)
        return carry

    lax.fori_loop(0, _EP // _NS // _CHUNK, body, 0)
    plsc.subcore_barrier()
    pltpu.sync_copy(acc.at[pl.ds(r0, _RPT)], out.at[c, pl.ds(r0, _RPT)])


@functools.cache
def _sc_scatter_kernel():
    return pl.kernel(
        _sc_scatter_body,
        out_type=jax.ShapeDtypeStruct((_NC, _NP, _H), jnp.float32),
        mesh=_sc_mesh(),
        compiler_params=pltpu.CompilerParams(use_tc_tiling_on_sc=False),
        scratch_types=[
            pltpu.VMEM((_CHUNK,), jnp.int32),
            pltpu.VMEM((_CHUNK,), jnp.int32),
            pltpu.VMEM((_CHUNK, _H), jnp.float32),
            pltpu.SemaphoreType.DMA,
            pltpu.VMEM_SHARED((_NP, _H), jnp.float32),
        ],
    )


def _sc_scatter(habf, src2, dstp, zra):
    return _sc_scatter_kernel()(habf, src2, dstp, zra)


def _sc_counts_body(onehot, dstp, zrows, out, didx_v, rows_v, acc):
    c = lax.axis_index("c")
    s = lax.axis_index("s")
    r0 = pl.multiple_of(s * _RPT, 8)
    pltpu.sync_copy(zrows, acc.at[pl.ds(r0, _RPT)])
    plsc.subcore_barrier()
    half = _EP // _NC
    tbase = c * half + s * (half // _NS)

    def body(i, carry):
        base = tbase + i * _CHUNK
        pltpu.sync_copy(onehot.at[pl.ds(base, _CHUNK)], rows_v)
        pltpu.sync_copy(dstp.at[pl.ds(base, _CHUNK)], didx_v)
        pltpu.sync_copy(rows_v, acc.at[didx_v], add=True)
        return carry

    lax.fori_loop(0, half // _NS // _CHUNK, body, 0)
    plsc.subcore_barrier()
    pltpu.sync_copy(acc.at[pl.ds(r0, _RPT)], out.at[c, pl.ds(r0, _RPT)])


@functools.cache
def _sc_counts_kernel():
    return pl.kernel(
        _sc_counts_body,
        out_type=jax.ShapeDtypeStruct((_NC, _NP, 16), jnp.float32),
        mesh=_sc_mesh(),
        compiler_params=pltpu.CompilerParams(use_tc_tiling_on_sc=False),
        scratch_types=[
            pltpu.VMEM((_CHUNK,), jnp.int32),
            pltpu.VMEM((_CHUNK, 16), jnp.float32),
            pltpu.VMEM_SHARED((_NP, 16), jnp.float32),
        ],
    )


def _sc_counts(onehot, dstp, zrows):
    return _sc_counts_kernel()(onehot, dstp, zrows)


def _tc_embed_body(x0, x1, xe1c, xe2c, h0):
    i128 = lax.broadcasted_iota(jnp.int32, (1, 128), 1)
    oh1 = (x0[...] == i128).astype(jnp.float32)
    i8 = lax.broadcasted_iota(jnp.int32, (1, 8), 1)
    oh2 = (x1[...] == i8).astype(jnp.float32)
    for c in range(_NC):
        h0[c] = (jnp.dot(oh1, xe1c[c], preferred_element_type=jnp.float32, precision=lax.Precision.HIGHEST)
                 + jnp.dot(oh2, xe2c[c], preferred_element_type=jnp.float32, precision=lax.Precision.HIGHEST))


def _tc_embed(x0, x1, xe1c, xe2c):
    return pl.pallas_call(
        _tc_embed_body,
        grid=(_N // _RB,),
        in_specs=[
            pl.BlockSpec((_RB, 1), lambda i: (i, 0)),
            pl.BlockSpec((_RB, 1), lambda i: (i, 0)),
            pl.BlockSpec((_NC, 128, _H), lambda i: (0, 0, 0)),
            pl.BlockSpec((_NC, 8, _H), lambda i: (0, 0, 0)),
        ],
        out_specs=pl.BlockSpec((_NC, _RB, _H), lambda i: (0, i, 0)),
        out_shape=jax.ShapeDtypeStruct((_NC, _N, _H), jnp.float32),
    )(x0, x1, xe1c, xe2c)


def _tc_onehot_body(ea0, ea1, ohE):
    i16 = lax.broadcasted_iota(jnp.int32, (1, 16), 1)
    ohE[...] = (ea0[...] == i16).astype(jnp.float32) + (
        (ea1[...] + 5) == i16
    ).astype(jnp.float32)


def _tc_onehot(ea0, ea1):
    return pl.pallas_call(
        _tc_onehot_body,
        grid=(_EP // _EB,),
        in_specs=[
            pl.BlockSpec((_EB, 1), lambda i: (i, 0)),
            pl.BlockSpec((_EB, 1), lambda i: (i, 0)),
        ],
        out_specs=pl.BlockSpec((_EB, 16), lambda i: (i, 0)),
        out_shape=jax.ShapeDtypeStruct((_EP, 16), jnp.float32),
    )(ea0, ea1)


def _tc_mm_body(S, h, cnt2, etabc, rrowc, w1c, b1r, w2c, b2c,
                h2_out, sum_out, sq_out):
    @pl.when(pl.program_id(0) == 0)
    def _():
        sum_out[...] = jnp.zeros_like(sum_out)
        sq_out[...] = jnp.zeros_like(sq_out)

    cnt = cnt2[0] + cnt2[1]
    tot = b1r[...]
    for c in range(_NC):
        aggr = (S[c] + h[c] + rrowc[c]
                + jnp.dot(cnt, etabc[c], preferred_element_type=jnp.float32, precision=lax.Precision.HIGHEST))
        tot = tot + jnp.dot(aggr, w1c[c], preferred_element_type=jnp.float32, precision=lax.Precision.DEFAULT)
    hm = jnp.maximum(tot, 0.0)
    for c in range(_NC):
        h2 = (jnp.dot(hm, w2c[c], preferred_element_type=jnp.float32, precision=lax.Precision.DEFAULT)
              + b2c[c])
        h2_out[c] = h2
        sum_out[c] += jnp.sum(h2, axis=0, keepdims=True)
        sq_out[c] += jnp.sum(h2 * h2, axis=0, keepdims=True)


def _tc_mm(S, h, cnt2, etabc, rrowc, w1c, b1r, w2c, b2c):
    return pl.pallas_call(
        _tc_mm_body,
        grid=(_N // _RB,),
        in_specs=[
            pl.BlockSpec((_NC, _RB, _H), lambda i: (0, i, 0)),
            pl.BlockSpec((_NC, _RB, _H), lambda i: (0, i, 0)),
            pl.BlockSpec((_NC, _RB, 16), lambda i: (0, i, 0)),
            pl.BlockSpec((_NC, 16, _H), lambda i: (0, 0, 0)),
            pl.BlockSpec((_NC, 1, _H), lambda i: (0, 0, 0)),
            pl.BlockSpec((_NC, _H, 640), lambda i: (0, 0, 0)),
            pl.BlockSpec((1, 640), lambda i: (0, 0)),
            pl.BlockSpec((_NC, 640, _H), lambda i: (0, 0, 0)),
            pl.BlockSpec((_NC, 1, _H), lambda i: (0, 0, 0)),
        ],
        out_specs=[
            pl.BlockSpec((_NC, _RB, _H), lambda i: (0, i, 0)),
            pl.BlockSpec((_NC, 1, _H), lambda i: (0, 0, 0)),
            pl.BlockSpec((_NC, 1, _H), lambda i: (0, 0, 0)),
        ],
        out_shape=[
            jax.ShapeDtypeStruct((_NC, _N, _H), jnp.float32),
            jax.ShapeDtypeStruct((_NC, 1, _H), jnp.float32),
            jax.ShapeDtypeStruct((_NC, 1, _H), jnp.float32),
        ],
    )(S, h, cnt2, etabc, rrowc, w1c, b1r, w2c, b2c)


def _rsqrt_exact(x):
    r = lax.rsqrt(x)
    r = r * (1.5 - 0.5 * x * r * r)
    r = r * (1.5 - 0.5 * x * r * r)
    return r


def _recip_exact(x):
    r = 1.0 / x
    r = r * (2.0 - x * r)
    r = r * (2.0 - x * r)
    return r


def _tc_bn_body(h2, sums, sqs, gc, bc, h_out, *, relu):
    for c in range(_NC):
        m = sums[c] * (1.0 / _N)
        v = sqs[c] * (1.0 / _N) - m * m
        hn = (h2[c] - m) * _rsqrt_exact(v + 1e-5) * gc[c] + bc[c]
        if relu:
            hn = jnp.maximum(hn, 0.0)
        h_out[c] = hn


def _tc_bn(h2, sums, sqs, gc, bc, relu):
    return pl.pallas_call(
        functools.partial(_tc_bn_body, relu=relu),
        grid=(_N // _RB,),
        in_specs=[
            pl.BlockSpec((_NC, _RB, _H), lambda i: (0, i, 0)),
            pl.BlockSpec((_NC, 1, _H), lambda i: (0, 0, 0)),
            pl.BlockSpec((_NC, 1, _H), lambda i: (0, 0, 0)),
            pl.BlockSpec((_NC, 1, _H), lambda i: (0, 0, 0)),
            pl.BlockSpec((_NC, 1, _H), lambda i: (0, 0, 0)),
        ],
        out_specs=pl.BlockSpec((_NC, _RB, _H), lambda i: (0, i, 0)),
        out_shape=jax.ShapeDtypeStruct((_NC, _N, _H), jnp.float32),
    )(h2, sums, sqs, gc, bc)


def _tc_final_body(h, batch_r, clus, featWc, featBr, outW1, outB1r,
                   outW2, outB2r, labelEp, hg_out, lab_out):
    iG = lax.broadcasted_iota(jnp.int32, (_G, 1), 0)
    pt = (batch_r[...] == iG).astype(jnp.float32)
    cnts = jnp.sum(pt, axis=1, keepdims=True)
    inv = _recip_exact(jnp.maximum(cnts, 1.0))
    hg = featBr[...]
    for c in range(_NC):
        sums = jnp.dot(pt, h[c], preferred_element_type=jnp.float32, precision=lax.Precision.HIGHEST)
        hg = hg + jnp.dot(sums * inv, featWc[c],
                          preferred_element_type=jnp.float32, precision=lax.Precision.DEFAULT)
    z = jnp.maximum(
        jnp.dot(hg, outW1[...], preferred_element_type=jnp.float32, precision=lax.Precision.DEFAULT)
        + outB1r[...], 0.0)
    hg_out[...] = (
        jnp.dot(z, outW2[...], preferred_element_type=jnp.float32, precision=lax.Precision.DEFAULT)
        + outB2r[...])
    i8 = lax.broadcasted_iota(jnp.int32, (1, 8), 1)
    ohc = (clus[...] == i8).astype(jnp.float32)
    lab_out[...] = jnp.dot(ohc, labelEp[...],
                           preferred_element_type=jnp.float32, precision=lax.Precision.HIGHEST)


def kernel(xe1, xe2, ee1, ee2, w1, b1, w2, b2, bn_g, bn_b, featW, featB,
           outW1, outB1, outW2, outB2, labelE, x, edge_index, edge_attr,
           batch, cluster_idx):
    f32, i32 = jnp.float32, jnp.int32

    def colsplit(w):
        pad = [(0, 0)] * (w.ndim - 1) + [(0, _H - _DH)]
        return jnp.stack([jnp.pad(w[..., :_DH], pad),
                          jnp.pad(w[..., _DH:], pad)])

    def rowsplit(w, kpad=0):
        pad = [(0, _H - _DH), (0, kpad)]
        return jnp.stack([jnp.pad(w[:_DH], pad), jnp.pad(w[_DH:], pad)])

    xe1c = colsplit(jnp.pad(xe1, [(0, 128 - xe1.shape[0]), (0, 0)]))
    xe2c = colsplit(jnp.pad(xe2, [(0, 8 - xe2.shape[0]), (0, 0)]))
    etabc = colsplit(jnp.pad(jnp.concatenate([ee1, ee2], axis=1),
                             [(0, 0), (0, 8), (0, 0)]))
    rrowc = colsplit(ee1[:, 4, :] + ee2[:, 0, :])[:, :, None, :]
    w1c = jnp.stack([
        jnp.pad(w1[:, :_DH, :], [(0, 0), (0, _H - _DH), (0, 40)]),
        jnp.pad(w1[:, _DH:, :], [(0, 0), (0, _H - _DH), (0, 40)]),
    ])
    b1p = jnp.pad(b1, [(0, 0), (0, 40)])[:, None, :]
    w2c = colsplit(jnp.pad(w2, [(0, 0), (0, 40), (0, 0)]))
    b2c = colsplit(b2)[:, :, None, :]
    gc = colsplit(bn_g)[:, :, None, :]
    bc = colsplit(bn_b)[:, :, None, :]
    featWc = rowsplit(featW)
    featBr = featB[None, :]
    outB1r = outB1[None, :]
    outB2r = outB2[None, :]
    labelEp = jnp.pad(labelE, [(0, 6), (0, 0)])

    pad = _EP - _E
    srcp = jnp.pad(edge_index[0].astype(i32), (0, pad))
    src2 = jnp.stack([srcp, srcp + _N])
    dstp = jnp.pad(edge_index[1].astype(i32), (0, pad),
                   constant_values=_TRASH)
    ea0 = jnp.pad(edge_attr[:, 0].astype(i32), (0, pad))[:, None]
    ea1 = jnp.pad(edge_attr[:, 1].astype(i32), (0, pad))[:, None]
    x0 = x[:, 0].astype(i32)[:, None]
    x1 = x[:, 1].astype(i32)[:, None]
    batch_r = batch.astype(i32)[None, :]
    clus = cluster_idx.astype(i32)[:, None]
    zra = jnp.zeros((_RPT, _H), f32)
    zr16 = jnp.zeros((_RPT, 16), f32)

    h = _tc_embed(x0, x1, xe1c, xe2c)
    ohE = _tc_onehot(ea0, ea1)
    cnts2 = _sc_counts(ohE, dstp, zr16)

    for l in range(_L):
        S = _sc_scatter(h.reshape(_NC * _N, _H), src2, dstp, zra)
        h2, sums, sqs = _tc_mm(S, h, cnts2, etabc[:, l], rrowc[:, l],
                               w1c[:, l], b1p[l], w2c[:, l], b2c[:, l])
        h = _tc_bn(h2, sums, sqs, gc[:, l], bc[:, l], l < _L - 1)

    hg, lab = pl.pallas_call(
        _tc_final_body,
        out_shape=[
            jax.ShapeDtypeStruct((_G, _FD // 2), jnp.float32),
            jax.ShapeDtypeStruct((_G, _FD), jnp.float32),
        ],
    )(h, batch_r, clus, featWc, featBr, outW1, outB1r, outW2, outB2r,
      labelEp)
    return (hg, lab)
